# baseline (speedup 1.0000x reference)
import functools

import jax
import jax.numpy as jnp
from jax import lax
from jax.experimental import pallas as pl
from jax.experimental.pallas import tpu as pltpu
from jax.experimental.pallas import tpu_sc as plsc

D = 32
K = 8192
NCB = 4
TB = 256

NC = 2
NS = 16
NW = NC * NS

_SC_PARAMS = pltpu.CompilerParams(needs_layout_passes=False,
                                  use_tc_tiling_on_sc=False)


def _fused_argmin(x, et2):
    mm2 = lax.dot_general(x, et2, (((1,), (0,)), ((), ())),
                          preferred_element_type=jnp.float32)
    x2 = jnp.sum(x * x, axis=1, keepdims=True)
    e2 = 0.25 * jnp.sum(et2 * et2, axis=0, keepdims=True)
    d2 = x2 + e2 - mm2
    return jnp.argmin(d2, axis=1).astype(jnp.int32)


def _stage0_body(x_ref, et2_ref, idx_ref):
    idx = _fused_argmin(x_ref[...], et2_ref[...])
    idx_ref[...] = idx.reshape(1, 1, TB)


def _tc_stage0(x, et2):
    nblk = x.shape[0] // TB
    return pl.pallas_call(
        _stage0_body,
        grid=(nblk,),
        in_specs=[
            pl.BlockSpec((TB, D), lambda i: (i, 0)),
            pl.BlockSpec((D, K), lambda i: (0, 0)),
        ],
        out_specs=pl.BlockSpec((1, 1, TB), lambda i: (i, 0, 0)),
        out_shape=jax.ShapeDtypeStruct((nblk, 1, TB), jnp.int32),
    )(x, et2)


def _stage_body(xp_ref, qp_ref, qaccp_ref, et2_ref,
                idx_ref, x_ref, qacc_ref, ssq_ref):
    qp = qp_ref[...]
    x = xp_ref[...] - qp
    x_ref[...] = x
    qacc_ref[...] = qaccp_ref[...] + qp

    @pl.when(pl.program_id(0) == 0)
    def _():
        ssq_ref[0, 0] = 0.0

    ssq_ref[0, 0] += jnp.sum(x * x)
    idx = _fused_argmin(x, et2_ref[...])
    idx_ref[...] = idx.reshape(1, 1, TB)


def _tc_stage(xp, qp, qaccp, et2):
    nblk = xp.shape[0] // TB
    return pl.pallas_call(
        _stage_body,
        grid=(nblk,),
        in_specs=[
            pl.BlockSpec((TB, D), lambda i: (i, 0)),
            pl.BlockSpec((TB, D), lambda i: (i, 0)),
            pl.BlockSpec((TB, D), lambda i: (i, 0)),
            pl.BlockSpec((D, K), lambda i: (0, 0)),
        ],
        out_specs=[
            pl.BlockSpec((1, 1, TB), lambda i: (i, 0, 0)),
            pl.BlockSpec((TB, D), lambda i: (i, 0)),
            pl.BlockSpec((TB, D), lambda i: (i, 0)),
            pl.BlockSpec(memory_space=pltpu.SMEM),
        ],
        out_shape=[
            jax.ShapeDtypeStruct((nblk, 1, TB), jnp.int32),
            jax.ShapeDtypeStruct(xp.shape, jnp.float32),
            jax.ShapeDtypeStruct(xp.shape, jnp.float32),
            jax.ShapeDtypeStruct((1, 1), jnp.float32),
        ],
    )(xp, qp, qaccp, et2)


def _tail_body(xp_ref, qp_ref, qaccp_ref, qt_ref, ssq_ref):
    qp = qp_ref[...]
    x = xp_ref[...] - qp
    qt_ref[...] = qaccp_ref[...] + qp

    @pl.when(pl.program_id(0) == 0)
    def _():
        ssq_ref[0, 0] = 0.0

    ssq_ref[0, 0] += jnp.sum(x * x)


def _tc_tail(xp, qp, qaccp):
    nblk = xp.shape[0] // TB
    return pl.pallas_call(
        _tail_body,
        grid=(nblk,),
        in_specs=[
            pl.BlockSpec((TB, D), lambda i: (i, 0)),
            pl.BlockSpec((TB, D), lambda i: (i, 0)),
            pl.BlockSpec((TB, D), lambda i: (i, 0)),
        ],
        out_specs=[
            pl.BlockSpec((TB, D), lambda i: (i, 0)),
            pl.BlockSpec(memory_space=pltpu.SMEM),
        ],
        out_shape=[
            jax.ShapeDtypeStruct(xp.shape, jnp.float32),
            jax.ShapeDtypeStruct((1, 1), jnp.float32),
        ],
    )(xp, qp, qaccp)


def _make_sc_gather(n_tokens):
    bpw = n_tokens // NW
    mesh = plsc.VectorSubcoreMesh(core_axis_name="c", subcore_axis_name="s")

    @functools.partial(
        pl.kernel,
        mesh=mesh,
        compiler_params=_SC_PARAMS,
        out_type=jax.ShapeDtypeStruct((n_tokens, D), jnp.float32),
        scratch_types=[
            pltpu.VMEM((bpw,), jnp.int32),
            pltpu.VMEM((bpw, D), jnp.float32),
            pltpu.SemaphoreType.DMA,
        ],
    )
    def sc_gather(emb_hbm, idx_hbm, q_hbm, idx_v, rows_v, sem):
        wid = lax.axis_index("s") * NC + lax.axis_index("c")
        base = wid * bpw
        pltpu.sync_copy(idx_hbm.at[pl.ds(base, bpw)], idx_v)
        pltpu.async_copy(emb_hbm.at[idx_v], rows_v, sem).wait()
        pltpu.sync_copy(rows_v, q_hbm.at[pl.ds(base, bpw)])

    return sc_gather


def _make_sc_flags(n_tokens):
    bpw = n_tokens // NW
    mesh = plsc.VectorSubcoreMesh(core_axis_name="c", subcore_axis_name="s")

    @functools.partial(
        pl.kernel,
        mesh=mesh,
        compiler_params=_SC_PARAMS,
        out_type=jax.ShapeDtypeStruct((NW, K), jnp.float32),
        scratch_types=[
            pltpu.VMEM((bpw,), jnp.int32),
            pltpu.VMEM((K,), jnp.float32),
        ],
    )
    def sc_flags(idx_hbm, fl_hbm, idx_v, flag_v):
        wid = lax.axis_index("s") * NC + lax.axis_index("c")
        base = wid * bpw
        pltpu.sync_copy(idx_hbm.at[pl.ds(base, bpw)], idx_v)

        zeros16 = jnp.zeros((16,), jnp.float32)
        def zbody(i, carry):
            flag_v[pl.ds(i * 16, 16)] = zeros16
            return carry
        lax.fori_loop(0, K // 16, zbody, 0)

        ones16 = jnp.ones((16,), jnp.float32)
        def sbody(i, carry):
            iv = idx_v[pl.ds(i * 16, 16)]
            plsc.store_scatter(flag_v, [iv], ones16)
            return carry
        lax.fori_loop(0, bpw // 16, sbody, 0)

        pltpu.sync_copy(flag_v, fl_hbm.at[wid])

    return sc_flags


def _final_body(fl_ref, s0_ref, s1_ref, s2_ref, s3_ref,
                loss_ref, util_ref, n_total):
    used = jnp.float32(0.0)
    for i in range(NCB):
        tot = jnp.sum(fl_ref[i], axis=0)
        used += jnp.sum((tot > 0.0).astype(jnp.float32))
    util_ref[0, 0] = used / (K * NCB)
    ssq = s0_ref[0, 0] + s1_ref[0, 0] + s2_ref[0, 0] + s3_ref[0, 0]
    loss_ref[0, 0] = ssq * 2.0 / n_total / NCB


def _tc_final(flags, s0, s1, s2, s3, n_total):
    body = functools.partial(_final_body, n_total=float(n_total))
    smem = pl.BlockSpec(memory_space=pltpu.SMEM)
    return pl.pallas_call(
        body,
        in_specs=[pl.BlockSpec((NCB, NW, K), lambda: (0, 0, 0)),
                  smem, smem, smem, smem],
        out_specs=[smem, smem],
        out_shape=[
            jax.ShapeDtypeStruct((1, 1), jnp.float32),
            jax.ShapeDtypeStruct((1, 1), jnp.float32),
        ],
    )(flags, s0, s1, s2, s3)


def kernel(z, embeds):
    bz, d, tz = z.shape
    n = bz * tz
    x0 = z.transpose(0, 2, 1).reshape(n, d)
    embeds_t2 = embeds.transpose(0, 2, 1) * 2.0
    sc_gather = _make_sc_gather(n)
    sc_flags = _make_sc_flags(n)

    idx0 = _tc_stage0(x0, embeds_t2[0])
    q0 = sc_gather(embeds[0], idx0.reshape(n))
    fl0 = sc_flags(idx0.reshape(n))

    qacc0 = jnp.zeros_like(x0)
    x, qacc = x0, qacc0
    idxs, flags, ssqs = [idx0], [fl0], []
    q = q0
    for i in range(1, NCB):
        idx, x, qacc, ssq = _tc_stage(x, q, qacc, embeds_t2[i])
        q = sc_gather(embeds[i], idx.reshape(n))
        idxs.append(idx)
        flags.append(sc_flags(idx.reshape(n)))
        ssqs.append(ssq)

    qt, ssq_last = _tc_tail(x, q, qacc)
    ssqs.append(ssq_last)

    loss, util = _tc_final(jnp.stack(flags), *ssqs, n_total=n * d)
    quantized_total = qt.reshape(bz, tz, d).transpose(0, 2, 1)
    all_indices = jnp.stack([ix.reshape(n) for ix in idxs]).reshape(NCB, bz, tz)
    return (quantized_total, all_indices, loss.reshape(()), util.reshape(()))


# speedup vs baseline: 1.0106x; 1.0106x over previous
import functools

import jax
import jax.numpy as jnp
from jax import lax
from jax.experimental import pallas as pl
from jax.experimental.pallas import tpu as pltpu
from jax.experimental.pallas import tpu_sc as plsc

D = 32
K = 8192
NCB = 4
TB = 256

NC = 2
NS = 16
NW = NC * NS

_SC_PARAMS = pltpu.CompilerParams(needs_layout_passes=False,
                                  use_tc_tiling_on_sc=False)


def _fused_argmin(x, et2):
    mm2 = lax.dot_general(x, et2, (((1,), (0,)), ((), ())),
                          preferred_element_type=jnp.float32)
    x2 = jnp.sum(x * x, axis=1, keepdims=True)
    e2 = 0.25 * jnp.sum(et2 * et2, axis=0, keepdims=True)
    d2 = x2 + e2 - mm2
    return jnp.argmin(d2, axis=1).astype(jnp.int32)


def _stage0_body(x_ref, et2_ref, idx_ref):
    idx = _fused_argmin(x_ref[...], et2_ref[...])
    idx_ref[...] = idx.reshape(1, 1, TB)


def _tc_stage0(x, et2):
    nblk = x.shape[0] // TB
    return pl.pallas_call(
        _stage0_body,
        grid=(nblk,),
        in_specs=[
            pl.BlockSpec((TB, D), lambda i: (i, 0)),
            pl.BlockSpec((D, K), lambda i: (0, 0)),
        ],
        out_specs=pl.BlockSpec((1, 1, TB), lambda i: (i, 0, 0)),
        out_shape=jax.ShapeDtypeStruct((nblk, 1, TB), jnp.int32),
    )(x, et2)


def _stage_body(xp_ref, qp_ref, et2_ref, idx_ref, x_ref, ssq_ref):
    x = xp_ref[...] - qp_ref[...]
    x_ref[...] = x

    @pl.when(pl.program_id(0) == 0)
    def _():
        ssq_ref[0, 0] = 0.0

    ssq_ref[0, 0] += jnp.sum(x * x)
    idx = _fused_argmin(x, et2_ref[...])
    idx_ref[...] = idx.reshape(1, 1, TB)


def _tc_stage(xp, qp, et2):
    nblk = xp.shape[0] // TB
    return pl.pallas_call(
        _stage_body,
        grid=(nblk,),
        in_specs=[
            pl.BlockSpec((TB, D), lambda i: (i, 0)),
            pl.BlockSpec((TB, D), lambda i: (i, 0)),
            pl.BlockSpec((D, K), lambda i: (0, 0)),
        ],
        out_specs=[
            pl.BlockSpec((1, 1, TB), lambda i: (i, 0, 0)),
            pl.BlockSpec((TB, D), lambda i: (i, 0)),
            pl.BlockSpec(memory_space=pltpu.SMEM),
        ],
        out_shape=[
            jax.ShapeDtypeStruct((nblk, 1, TB), jnp.int32),
            jax.ShapeDtypeStruct(xp.shape, jnp.float32),
            jax.ShapeDtypeStruct((1, 1), jnp.float32),
        ],
    )(xp, qp, et2)


def _tail_body(x0_ref, x3_ref, q3_ref, qt_ref, ssq_ref):
    q3 = q3_ref[...]
    x3 = x3_ref[...]
    qt_ref[...] = (x0_ref[...] - x3) + q3
    x4 = x3 - q3

    @pl.when(pl.program_id(0) == 0)
    def _():
        ssq_ref[0, 0] = 0.0

    ssq_ref[0, 0] += jnp.sum(x4 * x4)


def _tc_tail(x0, x3, q3):
    nblk = x0.shape[0] // TB
    return pl.pallas_call(
        _tail_body,
        grid=(nblk,),
        in_specs=[
            pl.BlockSpec((TB, D), lambda i: (i, 0)),
            pl.BlockSpec((TB, D), lambda i: (i, 0)),
            pl.BlockSpec((TB, D), lambda i: (i, 0)),
        ],
        out_specs=[
            pl.BlockSpec((TB, D), lambda i: (i, 0)),
            pl.BlockSpec(memory_space=pltpu.SMEM),
        ],
        out_shape=[
            jax.ShapeDtypeStruct(x0.shape, jnp.float32),
            jax.ShapeDtypeStruct((1, 1), jnp.float32),
        ],
    )(x0, x3, q3)


def _make_sc_stage(n_tokens):
    bpw = n_tokens // NW
    mesh = plsc.VectorSubcoreMesh(core_axis_name="c", subcore_axis_name="s")

    @functools.partial(
        pl.kernel,
        mesh=mesh,
        compiler_params=_SC_PARAMS,
        out_type=[
            jax.ShapeDtypeStruct((n_tokens, D), jnp.float32),
            jax.ShapeDtypeStruct((NW, K), jnp.float32),
        ],
        scratch_types=[
            pltpu.VMEM((bpw,), jnp.int32),
            pltpu.VMEM((bpw, D), jnp.float32),
            pltpu.VMEM((K,), jnp.float32),
            pltpu.SemaphoreType.DMA,
        ],
    )
    def sc_stage(emb_hbm, idx_hbm, q_hbm, fl_hbm, idx_v, rows_v, flag_v, sem):
        wid = lax.axis_index("s") * NC + lax.axis_index("c")
        base = wid * bpw
        pltpu.sync_copy(idx_hbm.at[pl.ds(base, bpw)], idx_v)
        gather = pltpu.async_copy(emb_hbm.at[idx_v], rows_v, sem)

        zeros16 = jnp.zeros((16,), jnp.float32)
        def zbody(i, carry):
            flag_v[pl.ds(i * 16, 16)] = zeros16
            return carry
        lax.fori_loop(0, K // 16, zbody, 0)

        ones16 = jnp.ones((16,), jnp.float32)
        def sbody(i, carry):
            iv = idx_v[pl.ds(i * 16, 16)]
            plsc.store_scatter(flag_v, [iv], ones16)
            return carry
        lax.fori_loop(0, bpw // 16, sbody, 0)

        pltpu.sync_copy(flag_v, fl_hbm.at[wid])
        gather.wait()
        pltpu.sync_copy(rows_v, q_hbm.at[pl.ds(base, bpw)])

    return sc_stage


def _final_body(fl_ref, s0_ref, s1_ref, s2_ref, s3_ref,
                loss_ref, util_ref, n_total):
    used = jnp.float32(0.0)
    for i in range(NCB):
        tot = jnp.sum(fl_ref[i], axis=0)
        used += jnp.sum((tot > 0.0).astype(jnp.float32))
    util_ref[0, 0] = used / (K * NCB)
    ssq = s0_ref[0, 0] + s1_ref[0, 0] + s2_ref[0, 0] + s3_ref[0, 0]
    loss_ref[0, 0] = ssq * 2.0 / n_total / NCB


def _tc_final(flags, s0, s1, s2, s3, n_total):
    body = functools.partial(_final_body, n_total=float(n_total))
    smem = pl.BlockSpec(memory_space=pltpu.SMEM)
    return pl.pallas_call(
        body,
        in_specs=[pl.BlockSpec((NCB, NW, K), lambda: (0, 0, 0)),
                  smem, smem, smem, smem],
        out_specs=[smem, smem],
        out_shape=[
            jax.ShapeDtypeStruct((1, 1), jnp.float32),
            jax.ShapeDtypeStruct((1, 1), jnp.float32),
        ],
    )(flags, s0, s1, s2, s3)


def kernel(z, embeds):
    bz, d, tz = z.shape
    n = bz * tz
    x0 = z.transpose(0, 2, 1).reshape(n, d)
    embeds_t2 = embeds.transpose(0, 2, 1) * 2.0
    sc_stage = _make_sc_stage(n)

    idx0 = _tc_stage0(x0, embeds_t2[0])
    q, fl0 = sc_stage(embeds[0], idx0.reshape(n))

    x = x0
    idxs, flags, ssqs = [idx0], [fl0], []
    for i in range(1, NCB):
        idx, x, ssq = _tc_stage(x, q, embeds_t2[i])
        q, fl = sc_stage(embeds[i], idx.reshape(n))
        idxs.append(idx)
        flags.append(fl)
        ssqs.append(ssq)

    qt, ssq_last = _tc_tail(x0, x, q)
    ssqs.append(ssq_last)

    loss, util = _tc_final(jnp.stack(flags), *ssqs, n_total=n * d)
    quantized_total = qt.reshape(bz, tz, d).transpose(0, 2, 1)
    all_indices = jnp.stack([ix.reshape(n) for ix in idxs]).reshape(NCB, bz, tz)
    return (quantized_total, all_indices, loss.reshape(()), util.reshape(()))


# speedup vs baseline: 1.0826x; 1.0713x over previous
import functools

import jax
import jax.numpy as jnp
from jax import lax
from jax.experimental import pallas as pl
from jax.experimental.pallas import tpu as pltpu
from jax.experimental.pallas import tpu_sc as plsc

D = 32
K = 8192
NCB = 4
TB = 512

NC = 2
NS = 16
NW = NC * NS

_SC_PARAMS = pltpu.CompilerParams(needs_layout_passes=False,
                                  use_tc_tiling_on_sc=False)


def _fused_argmin(x, et2):
    mm2 = lax.dot_general(x, et2, (((1,), (0,)), ((), ())),
                          preferred_element_type=jnp.float32)
    x2 = jnp.sum(x * x, axis=1, keepdims=True)
    e2 = 0.25 * jnp.sum(et2 * et2, axis=0, keepdims=True)
    d2 = x2 + e2 - mm2
    return jnp.argmin(d2, axis=1).astype(jnp.int32)


def _stage0_body(x_ref, et2_ref, idx_ref):
    idx = _fused_argmin(x_ref[...], et2_ref[...])
    idx_ref[...] = idx.reshape(1, 1, TB)


def _tc_stage0(x, et2):
    nblk = x.shape[0] // TB
    return pl.pallas_call(
        _stage0_body,
        grid=(nblk,),
        in_specs=[
            pl.BlockSpec((TB, D), lambda i: (i, 0)),
            pl.BlockSpec((D, K), lambda i: (0, 0)),
        ],
        out_specs=pl.BlockSpec((1, 1, TB), lambda i: (i, 0, 0)),
        out_shape=jax.ShapeDtypeStruct((nblk, 1, TB), jnp.int32),
    )(x, et2)


def _stage_body(xp_ref, qp_ref, et2_ref, idx_ref, x_ref, ssq_ref):
    x = xp_ref[...] - qp_ref[...]
    x_ref[...] = x

    @pl.when(pl.program_id(0) == 0)
    def _():
        ssq_ref[0, 0] = 0.0

    ssq_ref[0, 0] += jnp.sum(x * x)
    idx = _fused_argmin(x, et2_ref[...])
    idx_ref[...] = idx.reshape(1, 1, TB)


def _tc_stage(xp, qp, et2):
    nblk = xp.shape[0] // TB
    return pl.pallas_call(
        _stage_body,
        grid=(nblk,),
        in_specs=[
            pl.BlockSpec((TB, D), lambda i: (i, 0)),
            pl.BlockSpec((TB, D), lambda i: (i, 0)),
            pl.BlockSpec((D, K), lambda i: (0, 0)),
        ],
        out_specs=[
            pl.BlockSpec((1, 1, TB), lambda i: (i, 0, 0)),
            pl.BlockSpec((TB, D), lambda i: (i, 0)),
            pl.BlockSpec(memory_space=pltpu.SMEM),
        ],
        out_shape=[
            jax.ShapeDtypeStruct((nblk, 1, TB), jnp.int32),
            jax.ShapeDtypeStruct(xp.shape, jnp.float32),
            jax.ShapeDtypeStruct((1, 1), jnp.float32),
        ],
    )(xp, qp, et2)


def _tail_body(x0_ref, x3_ref, q3_ref, qt_ref, ssq_ref):
    q3 = q3_ref[...]
    x3 = x3_ref[...]
    qt_ref[...] = (x0_ref[...] - x3) + q3
    x4 = x3 - q3

    @pl.when(pl.program_id(0) == 0)
    def _():
        ssq_ref[0, 0] = 0.0

    ssq_ref[0, 0] += jnp.sum(x4 * x4)


def _tc_tail(x0, x3, q3):
    nblk = x0.shape[0] // TB
    return pl.pallas_call(
        _tail_body,
        grid=(nblk,),
        in_specs=[
            pl.BlockSpec((TB, D), lambda i: (i, 0)),
            pl.BlockSpec((TB, D), lambda i: (i, 0)),
            pl.BlockSpec((TB, D), lambda i: (i, 0)),
        ],
        out_specs=[
            pl.BlockSpec((TB, D), lambda i: (i, 0)),
            pl.BlockSpec(memory_space=pltpu.SMEM),
        ],
        out_shape=[
            jax.ShapeDtypeStruct(x0.shape, jnp.float32),
            jax.ShapeDtypeStruct((1, 1), jnp.float32),
        ],
    )(x0, x3, q3)


def _make_sc_stage(n_tokens):
    bpw = n_tokens // NW
    mesh = plsc.VectorSubcoreMesh(core_axis_name="c", subcore_axis_name="s")

    @functools.partial(
        pl.kernel,
        mesh=mesh,
        compiler_params=_SC_PARAMS,
        out_type=[
            jax.ShapeDtypeStruct((n_tokens, D), jnp.float32),
            jax.ShapeDtypeStruct((NW, K), jnp.float32),
        ],
        scratch_types=[
            pltpu.VMEM((bpw,), jnp.int32),
            pltpu.VMEM((bpw, D), jnp.float32),
            pltpu.VMEM((K,), jnp.float32),
            pltpu.SemaphoreType.DMA,
        ],
    )
    def sc_stage(emb_hbm, idx_hbm, q_hbm, fl_hbm, idx_v, rows_v, flag_v, sem):
        wid = lax.axis_index("s") * NC + lax.axis_index("c")
        base = wid * bpw
        pltpu.sync_copy(idx_hbm.at[pl.ds(base, bpw)], idx_v)
        gather = pltpu.async_copy(emb_hbm.at[idx_v], rows_v, sem)

        zeros16 = jnp.zeros((16,), jnp.float32)
        def zbody(i, carry):
            flag_v[pl.ds(i * 16, 16)] = zeros16
            return carry
        lax.fori_loop(0, K // 16, zbody, 0)

        ones16 = jnp.ones((16,), jnp.float32)
        def sbody(i, carry):
            iv = idx_v[pl.ds(i * 16, 16)]
            plsc.store_scatter(flag_v, [iv], ones16)
            return carry
        lax.fori_loop(0, bpw // 16, sbody, 0)

        pltpu.sync_copy(flag_v, fl_hbm.at[wid])
        gather.wait()
        pltpu.sync_copy(rows_v, q_hbm.at[pl.ds(base, bpw)])

    return sc_stage


def _final_body(fl_ref, s0_ref, s1_ref, s2_ref, s3_ref,
                loss_ref, util_ref, n_total):
    used = jnp.float32(0.0)
    for i in range(NCB):
        tot = jnp.sum(fl_ref[i], axis=0)
        used += jnp.sum((tot > 0.0).astype(jnp.float32))
    util_ref[0, 0] = used / (K * NCB)
    ssq = s0_ref[0, 0] + s1_ref[0, 0] + s2_ref[0, 0] + s3_ref[0, 0]
    loss_ref[0, 0] = ssq * 2.0 / n_total / NCB


def _tc_final(flags, s0, s1, s2, s3, n_total):
    body = functools.partial(_final_body, n_total=float(n_total))
    smem = pl.BlockSpec(memory_space=pltpu.SMEM)
    return pl.pallas_call(
        body,
        in_specs=[pl.BlockSpec((NCB, NW, K), lambda: (0, 0, 0)),
                  smem, smem, smem, smem],
        out_specs=[smem, smem],
        out_shape=[
            jax.ShapeDtypeStruct((1, 1), jnp.float32),
            jax.ShapeDtypeStruct((1, 1), jnp.float32),
        ],
    )(flags, s0, s1, s2, s3)


def kernel(z, embeds):
    bz, d, tz = z.shape
    n = bz * tz
    x0 = z.transpose(0, 2, 1).reshape(n, d)
    embeds_t2 = embeds.transpose(0, 2, 1) * 2.0
    sc_stage = _make_sc_stage(n)

    idx0 = _tc_stage0(x0, embeds_t2[0])
    q, fl0 = sc_stage(embeds[0], idx0.reshape(n))

    x = x0
    idxs, flags, ssqs = [idx0], [fl0], []
    for i in range(1, NCB):
        idx, x, ssq = _tc_stage(x, q, embeds_t2[i])
        q, fl = sc_stage(embeds[i], idx.reshape(n))
        idxs.append(idx)
        flags.append(fl)
        ssqs.append(ssq)

    qt, ssq_last = _tc_tail(x0, x, q)
    ssqs.append(ssq_last)

    loss, util = _tc_final(jnp.stack(flags), *ssqs, n_total=n * d)
    quantized_total = qt.reshape(bz, tz, d).transpose(0, 2, 1)
    all_indices = jnp.stack([ix.reshape(n) for ix in idxs]).reshape(NCB, bz, tz)
    return (quantized_total, all_indices, loss.reshape(()), util.reshape(()))


# speedup vs baseline: 1.1352x; 1.0485x over previous
import functools

import jax
import jax.numpy as jnp
from jax import lax
from jax.experimental import pallas as pl
from jax.experimental.pallas import tpu as pltpu
from jax.experimental.pallas import tpu_sc as plsc

D = 32
K = 8192
NCB = 4
TB = 1024

NC = 2
NS = 16
NW = NC * NS

_SC_PARAMS = pltpu.CompilerParams(needs_layout_passes=False,
                                  use_tc_tiling_on_sc=False)


def _fused_argmin(x, et2):
    mm2 = lax.dot_general(x, et2, (((1,), (0,)), ((), ())),
                          preferred_element_type=jnp.float32)
    x2 = jnp.sum(x * x, axis=1, keepdims=True)
    e2 = 0.25 * jnp.sum(et2 * et2, axis=0, keepdims=True)
    d2 = x2 + e2 - mm2
    return jnp.argmin(d2, axis=1).astype(jnp.int32)


def _stage0_body(x_ref, et2_ref, idx_ref):
    idx = _fused_argmin(x_ref[...], et2_ref[...])
    idx_ref[...] = idx.reshape(1, 1, TB)


def _tc_stage0(x, et2):
    nblk = x.shape[0] // TB
    return pl.pallas_call(
        _stage0_body,
        grid=(nblk,),
        in_specs=[
            pl.BlockSpec((TB, D), lambda i: (i, 0)),
            pl.BlockSpec((D, K), lambda i: (0, 0)),
        ],
        out_specs=pl.BlockSpec((1, 1, TB), lambda i: (i, 0, 0)),
        out_shape=jax.ShapeDtypeStruct((nblk, 1, TB), jnp.int32),
    )(x, et2)


def _stage_body(xp_ref, qp_ref, et2_ref, idx_ref, x_ref, ssq_ref):
    x = xp_ref[...] - qp_ref[...]
    x_ref[...] = x

    @pl.when(pl.program_id(0) == 0)
    def _():
        ssq_ref[0, 0] = 0.0

    ssq_ref[0, 0] += jnp.sum(x * x)
    idx = _fused_argmin(x, et2_ref[...])
    idx_ref[...] = idx.reshape(1, 1, TB)


def _tc_stage(xp, qp, et2):
    nblk = xp.shape[0] // TB
    return pl.pallas_call(
        _stage_body,
        grid=(nblk,),
        in_specs=[
            pl.BlockSpec((TB, D), lambda i: (i, 0)),
            pl.BlockSpec((TB, D), lambda i: (i, 0)),
            pl.BlockSpec((D, K), lambda i: (0, 0)),
        ],
        out_specs=[
            pl.BlockSpec((1, 1, TB), lambda i: (i, 0, 0)),
            pl.BlockSpec((TB, D), lambda i: (i, 0)),
            pl.BlockSpec(memory_space=pltpu.SMEM),
        ],
        out_shape=[
            jax.ShapeDtypeStruct((nblk, 1, TB), jnp.int32),
            jax.ShapeDtypeStruct(xp.shape, jnp.float32),
            jax.ShapeDtypeStruct((1, 1), jnp.float32),
        ],
    )(xp, qp, et2)


def _tail_body(x0_ref, x3_ref, q3_ref, qt_ref, ssq_ref):
    q3 = q3_ref[...]
    x3 = x3_ref[...]
    qt_ref[...] = (x0_ref[...] - x3) + q3
    x4 = x3 - q3

    @pl.when(pl.program_id(0) == 0)
    def _():
        ssq_ref[0, 0] = 0.0

    ssq_ref[0, 0] += jnp.sum(x4 * x4)


def _tc_tail(x0, x3, q3):
    nblk = x0.shape[0] // TB
    return pl.pallas_call(
        _tail_body,
        grid=(nblk,),
        in_specs=[
            pl.BlockSpec((TB, D), lambda i: (i, 0)),
            pl.BlockSpec((TB, D), lambda i: (i, 0)),
            pl.BlockSpec((TB, D), lambda i: (i, 0)),
        ],
        out_specs=[
            pl.BlockSpec((TB, D), lambda i: (i, 0)),
            pl.BlockSpec(memory_space=pltpu.SMEM),
        ],
        out_shape=[
            jax.ShapeDtypeStruct(x0.shape, jnp.float32),
            jax.ShapeDtypeStruct((1, 1), jnp.float32),
        ],
    )(x0, x3, q3)


def _make_sc_stage(n_tokens):
    bpw = n_tokens // NW
    mesh = plsc.VectorSubcoreMesh(core_axis_name="c", subcore_axis_name="s")

    @functools.partial(
        pl.kernel,
        mesh=mesh,
        compiler_params=_SC_PARAMS,
        out_type=[
            jax.ShapeDtypeStruct((n_tokens, D), jnp.float32),
            jax.ShapeDtypeStruct((NW, K), jnp.float32),
        ],
        scratch_types=[
            pltpu.VMEM((bpw,), jnp.int32),
            pltpu.VMEM((bpw, D), jnp.float32),
            pltpu.VMEM((K,), jnp.float32),
            pltpu.SemaphoreType.DMA,
        ],
    )
    def sc_stage(emb_hbm, idx_hbm, q_hbm, fl_hbm, idx_v, rows_v, flag_v, sem):
        wid = lax.axis_index("s") * NC + lax.axis_index("c")
        base = wid * bpw
        pltpu.sync_copy(idx_hbm.at[pl.ds(base, bpw)], idx_v)
        gather = pltpu.async_copy(emb_hbm.at[idx_v], rows_v, sem)

        zeros16 = jnp.zeros((16,), jnp.float32)
        def zbody(i, carry):
            flag_v[pl.ds(i * 16, 16)] = zeros16
            return carry
        lax.fori_loop(0, K // 16, zbody, 0)

        ones16 = jnp.ones((16,), jnp.float32)
        def sbody(i, carry):
            iv = idx_v[pl.ds(i * 16, 16)]
            plsc.store_scatter(flag_v, [iv], ones16)
            return carry
        lax.fori_loop(0, bpw // 16, sbody, 0)

        pltpu.sync_copy(flag_v, fl_hbm.at[wid])
        gather.wait()
        pltpu.sync_copy(rows_v, q_hbm.at[pl.ds(base, bpw)])

    return sc_stage


def _final_body(fl_ref, s0_ref, s1_ref, s2_ref, s3_ref,
                loss_ref, util_ref, n_total):
    used = jnp.float32(0.0)
    for i in range(NCB):
        tot = jnp.sum(fl_ref[i], axis=0)
        used += jnp.sum((tot > 0.0).astype(jnp.float32))
    util_ref[0, 0] = used / (K * NCB)
    ssq = s0_ref[0, 0] + s1_ref[0, 0] + s2_ref[0, 0] + s3_ref[0, 0]
    loss_ref[0, 0] = ssq * 2.0 / n_total / NCB


def _tc_final(flags, s0, s1, s2, s3, n_total):
    body = functools.partial(_final_body, n_total=float(n_total))
    smem = pl.BlockSpec(memory_space=pltpu.SMEM)
    return pl.pallas_call(
        body,
        in_specs=[pl.BlockSpec((NCB, NW, K), lambda: (0, 0, 0)),
                  smem, smem, smem, smem],
        out_specs=[smem, smem],
        out_shape=[
            jax.ShapeDtypeStruct((1, 1), jnp.float32),
            jax.ShapeDtypeStruct((1, 1), jnp.float32),
        ],
    )(flags, s0, s1, s2, s3)


def kernel(z, embeds):
    bz, d, tz = z.shape
    n = bz * tz
    x0 = z.transpose(0, 2, 1).reshape(n, d)
    embeds_t2 = embeds.transpose(0, 2, 1) * 2.0
    sc_stage = _make_sc_stage(n)

    idx0 = _tc_stage0(x0, embeds_t2[0])
    q, fl0 = sc_stage(embeds[0], idx0.reshape(n))

    x = x0
    idxs, flags, ssqs = [idx0], [fl0], []
    for i in range(1, NCB):
        idx, x, ssq = _tc_stage(x, q, embeds_t2[i])
        q, fl = sc_stage(embeds[i], idx.reshape(n))
        idxs.append(idx)
        flags.append(fl)
        ssqs.append(ssq)

    qt, ssq_last = _tc_tail(x0, x, q)
    ssqs.append(ssq_last)

    loss, util = _tc_final(jnp.stack(flags), *ssqs, n_total=n * d)
    quantized_total = qt.reshape(bz, tz, d).transpose(0, 2, 1)
    all_indices = jnp.stack([ix.reshape(n) for ix in idxs]).reshape(NCB, bz, tz)
    return (quantized_total, all_indices, loss.reshape(()), util.reshape(()))
